# baseline (speedup 1.0000x reference)
import functools

import jax
import jax.numpy as jnp
from jax import lax
from jax.experimental import pallas as pl
from jax.experimental.pallas import tpu as pltpu
from jax.experimental.pallas import tpu_sc as plsc

V = 2048
E = 65536
C = 128
CE = 4
H = 128

NW = 32
RPW = V // NW
CHUNK = 128
NT = C // 16


def _proj_body(nf_ref, w_ref, b_ref, a_ref, b2_ref):
    nf = nf_ref[...]
    w1 = w_ref[0:C, :]
    w2 = w_ref[C:2 * C, :]
    a_ref[...] = (
        jnp.dot(nf, w1 - w2, preferred_element_type=jnp.float32) + b_ref[...]
    )
    b2_ref[...] = jnp.dot(nf, w2, preferred_element_type=jnp.float32)


_proj = pl.pallas_call(
    _proj_body,
    out_shape=(
        jax.ShapeDtypeStruct((V, C), jnp.float32),
        jax.ShapeDtypeStruct((V, C), jnp.float32),
    ),
)


_sc_mesh = plsc.VectorSubcoreMesh(
    core_axis_name="c", subcore_axis_name="s", num_cores=2, num_subcores=16)


_SC_SCRATCH = [
    pltpu.VMEM((RPW + 16,), jnp.int32),
    pltpu.VMEM((CHUNK + 24,), jnp.int32),
    pltpu.VMEM((CHUNK + 24,), jnp.int32),
    pltpu.VMEM((CHUNK,), jnp.int32),
    pltpu.VMEM((CHUNK,), jnp.int32),
    pltpu.VMEM((CHUNK, C), jnp.float32),
    pltpu.VMEM((CHUNK, 16), jnp.float32),
    pltpu.VMEM((RPW, C), jnp.float32),
    pltpu.VMEM((CE, C), jnp.float32),
    pltpu.VMEM((RPW, C), jnp.float32),
    pltpu.SemaphoreType.DMA,
]


def _sc_body(rs_hbm, src_hbm, dst_hbm, a_hbm, b2_hbm, e4_hbm, w3_hbm,
                    out_hbm, rs_v, srcb, dstb, idxb, idxe, brows, erows,
                    arows, w3v, acc, sem):
    wid = lax.axis_index("s") * 2 + lax.axis_index("c")
    r0 = wid * RPW

    pltpu.sync_copy(rs_hbm.at[pl.ds(r0, RPW + 16)], rs_v)
    pltpu.sync_copy(a_hbm.at[pl.ds(r0, RPW)], arows)
    pltpu.sync_copy(w3_hbm, w3v)

    neg = jnp.full((16,), -jnp.inf, dtype=jnp.float32)

    def init_body(i, _):
        r = i // NT
        t = i % NT
        acc[r, pl.ds(t * 16, 16)] = neg
        return 0

    lax.fori_loop(0, RPW * NT, init_body, 0)

    s = rs_v[pl.ds(0, 16)][0]
    cnt = rs_v[pl.ds(RPW, 16)][0] - s
    off = s % 8
    sa = s - off
    nchunks = (cnt + CHUNK - 1) // CHUNK

    def chunk_body(k, _):
        base = pl.multiple_of(sa + k * CHUNK, 8)
        pltpu.sync_copy(src_hbm.at[pl.ds(base, CHUNK + 8)],
                        srcb.at[pl.ds(0, CHUNK + 8)])
        pltpu.sync_copy(dst_hbm.at[pl.ds(base, CHUNK + 8)],
                        dstb.at[pl.ds(0, CHUNK + 8)])
        lanes = jnp.arange(16, dtype=jnp.int32)
        for t in range(CHUNK // 16):
            ii = lanes + (off + t * 16)
            sv = plsc.load_gather(srcb, [ii])
            dv = plsc.load_gather(dstb, [ii])
            idxb[pl.ds(t * 16, 16)] = dv
            idxe[pl.ds(t * 16, 16)] = (sv * V + dv) // 4
        pltpu.async_copy(b2_hbm.at[idxb], brows, sem).wait()
        pltpu.async_copy(e4_hbm.at[idxe], erows, sem).wait()
        n = jnp.minimum(cnt - k * CHUNK, CHUNK)

        def edge_body(j, _):
            r = srcb[pl.ds(off + j, 16)][0] - r0
            q = (dstb[pl.ds(off + j, 16)][0] & 3) * 4
            jf = jnp.full((16,), j, dtype=jnp.int32)
            e0 = plsc.load_gather(erows, [jf, jnp.full((16,), 0, jnp.int32) + q])
            e1 = plsc.load_gather(erows, [jf, jnp.full((16,), 1, jnp.int32) + q])
            e2 = plsc.load_gather(erows, [jf, jnp.full((16,), 2, jnp.int32) + q])
            e3 = plsc.load_gather(erows, [jf, jnp.full((16,), 3, jnp.int32) + q])
            for t in range(NT):
                cs = pl.ds(t * 16, 16)
                v = arows[r, cs] + brows[j, cs]
                v = v + e0 * w3v[0, cs] + e1 * w3v[1, cs]
                v = v + e2 * w3v[2, cs] + e3 * w3v[3, cs]
                v = jnp.maximum(v, 0.0)
                acc[r, cs] = jnp.maximum(acc[r, cs], v)
            return 0

        lax.fori_loop(0, n, edge_body, 0)
        return 0

    lax.fori_loop(0, nchunks, chunk_body, 0)

    pltpu.sync_copy(acc, out_hbm.at[pl.ds(r0, RPW)])


_sc_edge_kernel = pl.kernel(
    _sc_body,
    out_type=jax.ShapeDtypeStruct((V, C), jnp.float32),
    mesh=_sc_mesh,
    compiler_params=pltpu.CompilerParams(
        needs_layout_passes=False, use_tc_tiling_on_sc=False),
    scratch_types=_SC_SCRATCH,
)


def kernel(adj_src, adj_dst, adj_values, node_features, edge_attributes, W, b):
    del adj_values
    a_proj, b_proj = _proj(node_features, W, b.reshape(1, C))
    rs = jnp.searchsorted(
        adj_src, jnp.arange(V + 1, dtype=jnp.int32)
    ).astype(jnp.int32)
    rs = jnp.concatenate([rs, jnp.full((15,), E, dtype=jnp.int32)])
    pad = jnp.zeros((CHUNK + 8,), dtype=jnp.int32)
    src_p = jnp.concatenate([adj_src, pad])
    dst_p = jnp.concatenate([adj_dst, pad])
    e4 = edge_attributes.reshape(V * V * CE // 16, 16)
    w3 = W[2 * C:, :]
    return _sc_edge_kernel(rs, src_p, dst_p, a_proj, b_proj, e4, w3)


# speedup vs baseline: 11.1464x; 11.1464x over previous
import functools

import jax
import jax.numpy as jnp
from jax import lax
from jax.experimental import pallas as pl
from jax.experimental.pallas import tpu as pltpu
from jax.experimental.pallas import tpu_sc as plsc

V = 2048
E = 65536
C = 128
CE = 4
H = 128

NW = 32
RPW = V // NW
CHUNK = 128
NT = C // 16


def _proj_body(nf_ref, w_ref, b_ref, a_ref, b2_ref):
    nf = nf_ref[...]
    w1 = w_ref[0:C, :]
    w2 = w_ref[C:2 * C, :]
    a_ref[...] = (
        jnp.dot(nf, w1 - w2, preferred_element_type=jnp.float32) + b_ref[...]
    )
    b2_ref[...] = jnp.dot(nf, w2, preferred_element_type=jnp.float32)


_proj = pl.pallas_call(
    _proj_body,
    out_shape=(
        jax.ShapeDtypeStruct((V, C), jnp.float32),
        jax.ShapeDtypeStruct((V, C), jnp.float32),
    ),
)


_sc_mesh = plsc.VectorSubcoreMesh(
    core_axis_name="c", subcore_axis_name="s", num_cores=2, num_subcores=16)


_SC_SCRATCH = [
    pltpu.VMEM((RPW + 16,), jnp.int32),
    pltpu.VMEM((CHUNK + 24,), jnp.int32),
    pltpu.VMEM((CHUNK + 24,), jnp.int32),
    pltpu.VMEM((CHUNK,), jnp.int32),
    pltpu.VMEM((CHUNK,), jnp.int32),
    pltpu.VMEM((CHUNK, C), jnp.float32),
    pltpu.VMEM((CHUNK, 512), jnp.float32),
    pltpu.VMEM((RPW, C), jnp.float32),
    pltpu.VMEM((CE, C), jnp.float32),
    pltpu.VMEM((RPW, C), jnp.float32),
    pltpu.SemaphoreType.DMA,
]


def _sc_body(rs_hbm, src_hbm, dst_hbm, a_hbm, b2_hbm, e4_hbm, w3_hbm,
                    out_hbm, rs_v, srcb, dstb, idxb, idxe, brows, erows,
                    arows, w3v, acc, sem):
    wid = lax.axis_index("s") * 2 + lax.axis_index("c")
    r0 = wid * RPW

    pltpu.sync_copy(rs_hbm.at[pl.ds(r0, RPW + 16)], rs_v)
    pltpu.sync_copy(a_hbm.at[pl.ds(r0, RPW)], arows)
    pltpu.sync_copy(w3_hbm, w3v)

    neg = jnp.full((16,), -jnp.inf, dtype=jnp.float32)

    def init_body(i, _):
        r = i // NT
        t = i % NT
        acc[r, pl.ds(t * 16, 16)] = neg
        return 0

    lax.fori_loop(0, RPW * NT, init_body, 0)

    s = rs_v[pl.ds(0, 16)][0]
    cnt = rs_v[pl.ds(RPW, 16)][0] - s
    off = s % 8
    sa = s - off
    nchunks = (cnt + CHUNK - 1) // CHUNK

    def chunk_body(k, _):
        base = pl.multiple_of(sa + k * CHUNK, 8)
        pltpu.sync_copy(src_hbm.at[pl.ds(base, CHUNK + 8)],
                        srcb.at[pl.ds(0, CHUNK + 8)])
        pltpu.sync_copy(dst_hbm.at[pl.ds(base, CHUNK + 8)],
                        dstb.at[pl.ds(0, CHUNK + 8)])
        lanes = jnp.arange(16, dtype=jnp.int32)
        for t in range(CHUNK // 16):
            ii = lanes + (off + t * 16)
            sv = plsc.load_gather(srcb, [ii])
            dv = plsc.load_gather(dstb, [ii])
            idxb[pl.ds(t * 16, 16)] = dv
            idxe[pl.ds(t * 16, 16)] = sv * 16 + (dv >> 7)
        pltpu.async_copy(b2_hbm.at[idxb], brows, sem).wait()
        pltpu.async_copy(e4_hbm.at[idxe], erows, sem).wait()
        n = jnp.minimum(cnt - k * CHUNK, CHUNK)

        def edge_body(j, _):
            r = srcb[pl.ds(off + j, 16)][0] - r0
            ci = dstb[pl.ds(off + j, 16)][0] & 127
            jf = jnp.full((16,), j, dtype=jnp.int32)
            e0 = plsc.load_gather(erows, [jf, jnp.full((16,), 0, jnp.int32) + ci])
            e1 = plsc.load_gather(erows, [jf, jnp.full((16,), 128, jnp.int32) + ci])
            e2 = plsc.load_gather(erows, [jf, jnp.full((16,), 256, jnp.int32) + ci])
            e3 = plsc.load_gather(erows, [jf, jnp.full((16,), 384, jnp.int32) + ci])
            for t in range(NT):
                cs = pl.ds(t * 16, 16)
                v = arows[r, cs] + brows[j, cs]
                v = v + e0 * w3v[0, cs] + e1 * w3v[1, cs]
                v = v + e2 * w3v[2, cs] + e3 * w3v[3, cs]
                v = jnp.maximum(v, 0.0)
                acc[r, cs] = jnp.maximum(acc[r, cs], v)
            return 0

        lax.fori_loop(0, n, edge_body, 0)
        return 0

    lax.fori_loop(0, nchunks, chunk_body, 0)

    pltpu.sync_copy(acc, out_hbm.at[pl.ds(r0, RPW)])


_sc_edge_kernel = pl.kernel(
    _sc_body,
    out_type=jax.ShapeDtypeStruct((V, C), jnp.float32),
    mesh=_sc_mesh,
    compiler_params=pltpu.CompilerParams(
        needs_layout_passes=False, use_tc_tiling_on_sc=False),
    scratch_types=_SC_SCRATCH,
)


def kernel(adj_src, adj_dst, adj_values, node_features, edge_attributes, W, b):
    del adj_values
    a_proj, b_proj = _proj(node_features, W, b.reshape(1, C))
    rs = jnp.searchsorted(
        adj_src, jnp.arange(V + 1, dtype=jnp.int32)
    ).astype(jnp.int32)
    rs = jnp.concatenate([rs, jnp.full((15,), E, dtype=jnp.int32)])
    pad = jnp.zeros((CHUNK + 8,), dtype=jnp.int32)
    src_p = jnp.concatenate([adj_src, pad])
    dst_p = jnp.concatenate([adj_dst, pad])
    e4 = edge_attributes.reshape(1, V, 16, 128, CE).transpose(
        0, 1, 2, 4, 3).reshape(V * 16, CE * 128)
    w3 = W[2 * C:, :]
    return _sc_edge_kernel(rs, src_p, dst_p, a_proj, b_proj, e4, w3)


# speedup vs baseline: 11.9427x; 1.0714x over previous
import functools

import jax
import jax.numpy as jnp
from jax import lax
from jax.experimental import pallas as pl
from jax.experimental.pallas import tpu as pltpu
from jax.experimental.pallas import tpu_sc as plsc

V = 2048
E = 65536
C = 128
CE = 4
H = 128

NW = 32
RPW = V // NW
CHUNK = 128
NT = C // 16


def _proj_body(nf_ref, w_ref, b_ref, a_ref, b2_ref):
    nf = nf_ref[...]
    w1 = w_ref[0:C, :]
    w2 = w_ref[C:2 * C, :]
    a_ref[...] = (
        jnp.dot(nf, w1 - w2, preferred_element_type=jnp.float32) + b_ref[...]
    )
    b2_ref[...] = jnp.dot(nf, w2, preferred_element_type=jnp.float32)


_proj = pl.pallas_call(
    _proj_body,
    out_shape=(
        jax.ShapeDtypeStruct((V, C), jnp.float32),
        jax.ShapeDtypeStruct((V, C), jnp.float32),
    ),
)


_sc_mesh = plsc.VectorSubcoreMesh(
    core_axis_name="c", subcore_axis_name="s", num_cores=2, num_subcores=16)


_SC_SCRATCH = [
    pltpu.VMEM((RPW + 16,), jnp.int32),
    pltpu.VMEM((CHUNK + 24,), jnp.int32),
    pltpu.VMEM((CHUNK + 24,), jnp.int32),
    pltpu.VMEM((CHUNK,), jnp.int32),
    pltpu.VMEM((CHUNK,), jnp.int32),
    pltpu.VMEM((CHUNK, C), jnp.float32),
    pltpu.VMEM((CHUNK, 512), jnp.float32),
    pltpu.VMEM((RPW, C), jnp.float32),
    pltpu.VMEM((CE, C), jnp.float32),
    pltpu.VMEM((RPW, C), jnp.float32),
    pltpu.SemaphoreType.DMA,
    pltpu.SemaphoreType.DMA,
]


def _sc_body(rs_hbm, src_hbm, dst_hbm, a_hbm, b2_hbm, e4_hbm, w3_hbm,
                    out_hbm, rs_v, srcb, dstb, idxb, idxe, brows, erows,
                    arows, w3v, acc, sem, sem2):
    wid = lax.axis_index("s") * 2 + lax.axis_index("c")
    r0 = wid * RPW

    pltpu.sync_copy(rs_hbm.at[pl.ds(r0, RPW + 16)], rs_v)
    pltpu.sync_copy(a_hbm.at[pl.ds(r0, RPW)], arows)
    pltpu.sync_copy(w3_hbm, w3v)
    w3r = [[w3v[k, pl.ds(t * 16, 16)] for t in range(NT)] for k in range(CE)]

    neg = jnp.full((16,), -jnp.inf, dtype=jnp.float32)

    def init_body(i, _):
        r = i // NT
        t = i % NT
        acc[r, pl.ds(t * 16, 16)] = neg
        return 0

    lax.fori_loop(0, RPW * NT, init_body, 0)

    s = rs_v[pl.ds(0, 16)][0]
    cnt = rs_v[pl.ds(RPW, 16)][0] - s
    off = s % 8
    sa = s - off
    nchunks = (cnt + CHUNK - 1) // CHUNK

    def chunk_body(k, _):
        base = pl.multiple_of(sa + k * CHUNK, 8)
        pltpu.sync_copy(src_hbm.at[pl.ds(base, CHUNK + 8)],
                        srcb.at[pl.ds(0, CHUNK + 8)])
        pltpu.sync_copy(dst_hbm.at[pl.ds(base, CHUNK + 8)],
                        dstb.at[pl.ds(0, CHUNK + 8)])
        lanes = jnp.arange(16, dtype=jnp.int32)
        for t in range(CHUNK // 16):
            ii = lanes + (off + t * 16)
            sv = plsc.load_gather(srcb, [ii])
            dv = plsc.load_gather(dstb, [ii])
            idxb[pl.ds(t * 16, 16)] = dv
            idxe[pl.ds(t * 16, 16)] = sv * 16 + (dv >> 7)
        cpb = pltpu.async_copy(b2_hbm.at[idxb], brows, sem)
        cpe = pltpu.async_copy(e4_hbm.at[idxe], erows, sem2)
        cpb.wait()
        cpe.wait()
        n = jnp.minimum(cnt - k * CHUNK, CHUNK)

        def edge_body(j, _):
            r = srcb[pl.ds(off + j, 16)][0] - r0
            ci = dstb[pl.ds(off + j, 16)][0] & 127
            jf = jnp.full((16,), j, dtype=jnp.int32)
            e0 = plsc.load_gather(erows, [jf, jnp.full((16,), 0, jnp.int32) + ci])
            e1 = plsc.load_gather(erows, [jf, jnp.full((16,), 128, jnp.int32) + ci])
            e2 = plsc.load_gather(erows, [jf, jnp.full((16,), 256, jnp.int32) + ci])
            e3 = plsc.load_gather(erows, [jf, jnp.full((16,), 384, jnp.int32) + ci])
            for t in range(NT):
                cs = pl.ds(t * 16, 16)
                v = brows[j, cs] + e0 * w3r[0][t]
                v = v + e1 * w3r[1][t] + e2 * w3r[2][t] + e3 * w3r[3][t]
                acc[r, cs] = jnp.maximum(acc[r, cs], v)
            return 0

        lax.fori_loop(0, n, edge_body, 0)
        return 0

    lax.fori_loop(0, nchunks, chunk_body, 0)

    def fin_body(i, _):
        r = i // NT
        t = i % NT
        cs = pl.ds(t * 16, 16)
        v = acc[r, cs]
        res = jnp.maximum(arows[r, cs] + v, 0.0)
        acc[r, cs] = jnp.where(v == neg, v, res)
        return 0

    lax.fori_loop(0, RPW * NT, fin_body, 0)

    pltpu.sync_copy(acc, out_hbm.at[pl.ds(r0, RPW)])


_sc_edge_kernel = pl.kernel(
    _sc_body,
    out_type=jax.ShapeDtypeStruct((V, C), jnp.float32),
    mesh=_sc_mesh,
    compiler_params=pltpu.CompilerParams(
        needs_layout_passes=False, use_tc_tiling_on_sc=False),
    scratch_types=_SC_SCRATCH,
)


def kernel(adj_src, adj_dst, adj_values, node_features, edge_attributes, W, b):
    del adj_values
    a_proj, b_proj = _proj(node_features, W, b.reshape(1, C))
    rs = jnp.searchsorted(
        adj_src, jnp.arange(V + 1, dtype=jnp.int32)
    ).astype(jnp.int32)
    rs = jnp.concatenate([rs, jnp.full((15,), E, dtype=jnp.int32)])
    pad = jnp.zeros((CHUNK + 8,), dtype=jnp.int32)
    src_p = jnp.concatenate([adj_src, pad])
    dst_p = jnp.concatenate([adj_dst, pad])
    e4 = edge_attributes.reshape(1, V, 16, 128, CE).transpose(
        0, 1, 2, 4, 3).reshape(V * 16, CE * 128)
    w3 = W[2 * C:, :]
    return _sc_edge_kernel(rs, src_p, dst_p, a_proj, b_proj, e4, w3)


# speedup vs baseline: 15.5523x; 1.3022x over previous
import functools

import jax
import jax.numpy as jnp
from jax import lax
from jax.experimental import pallas as pl
from jax.experimental.pallas import tpu as pltpu
from jax.experimental.pallas import tpu_sc as plsc

V = 2048
E = 65536
C = 128
CE = 4
H = 128

NW = 32
RPW = V // NW
CHUNK = 128
NT = C // 16


def _proj_body(nf_ref, w_ref, b_ref, a_ref, b2_ref):
    nf = nf_ref[...]
    w1 = w_ref[0:C, :]
    w2 = w_ref[C:2 * C, :]
    a_ref[...] = (
        jnp.dot(nf, w1 - w2, preferred_element_type=jnp.float32) + b_ref[...]
    )
    b2_ref[...] = jnp.dot(nf, w2, preferred_element_type=jnp.float32)


_proj = pl.pallas_call(
    _proj_body,
    out_shape=(
        jax.ShapeDtypeStruct((V, C), jnp.float32),
        jax.ShapeDtypeStruct((V, C), jnp.float32),
    ),
)


_sc_mesh = plsc.VectorSubcoreMesh(
    core_axis_name="c", subcore_axis_name="s", num_cores=2, num_subcores=16)


_SC_SCRATCH = [
    pltpu.VMEM((RPW + 16,), jnp.int32),
    pltpu.VMEM((CHUNK + 24,), jnp.int32),
    pltpu.VMEM((CHUNK + 24,), jnp.int32),
    pltpu.VMEM((CHUNK,), jnp.int32),
    pltpu.VMEM((CHUNK,), jnp.int32),
    pltpu.VMEM((CHUNK, C), jnp.float32),
    pltpu.VMEM((CHUNK, 512), jnp.float32),
    pltpu.VMEM((RPW, C), jnp.float32),
    pltpu.VMEM((CE, C), jnp.float32),
    pltpu.VMEM((RPW, C), jnp.float32),
    pltpu.SemaphoreType.DMA,
    pltpu.SemaphoreType.DMA,
]


def _sc_body(rs_hbm, src_hbm, dst_hbm, a_hbm, b2_hbm, e4_hbm, w3_hbm,
                    out_hbm, rs_v, srcb, dstb, idxb, idxe, brows, erows,
                    arows, w3v, acc, sem, sem2):
    wid = lax.axis_index("s") * 2 + lax.axis_index("c")
    r0 = wid * RPW

    pltpu.sync_copy(rs_hbm.at[pl.ds(r0, RPW + 16)], rs_v)
    pltpu.sync_copy(a_hbm.at[pl.ds(r0, RPW)], arows)
    pltpu.sync_copy(w3_hbm, w3v)
    w3r = [[w3v[k, pl.ds(t * 16, 16)] for t in range(NT)] for k in range(CE)]

    neg = jnp.full((16,), -jnp.inf, dtype=jnp.float32)

    def init_body(i, _):
        r = i // NT
        t = i % NT
        acc[r, pl.ds(t * 16, 16)] = neg
        return 0

    lax.fori_loop(0, RPW * NT, init_body, 0)

    s = rs_v[pl.ds(0, 16)][0]
    cnt = rs_v[pl.ds(RPW, 16)][0] - s
    off = s % 8
    sa = s - off
    nchunks = (cnt + CHUNK - 1) // CHUNK

    def chunk_body(k, _):
        base = pl.multiple_of(sa + k * CHUNK, 8)
        pltpu.sync_copy(src_hbm.at[pl.ds(base, CHUNK + 8)],
                        srcb.at[pl.ds(0, CHUNK + 8)])
        pltpu.sync_copy(dst_hbm.at[pl.ds(base, CHUNK + 8)],
                        dstb.at[pl.ds(0, CHUNK + 8)])
        lanes = jnp.arange(16, dtype=jnp.int32)
        for t in range(CHUNK // 16):
            ii = lanes + (off + t * 16)
            sv = plsc.load_gather(srcb, [ii])
            dv = plsc.load_gather(dstb, [ii])
            idxb[pl.ds(t * 16, 16)] = dv
            idxe[pl.ds(t * 16, 16)] = sv * 16 + (dv >> 7)
        cpb = pltpu.async_copy(b2_hbm.at[idxb], brows, sem)
        cpe = pltpu.async_copy(e4_hbm.at[idxe], erows, sem2)
        cpb.wait()
        cpe.wait()
        n = jnp.minimum(cnt - k * CHUNK, CHUNK)

        def edge_body(j, _):
            r = srcb[pl.ds(off + j, 16)][0] - r0
            ci = dstb[pl.ds(off + j, 16)][0] & 127
            jf = jnp.full((16,), j, dtype=jnp.int32)
            e0 = plsc.load_gather(erows, [jf, jnp.full((16,), 0, jnp.int32) + ci])
            e1 = plsc.load_gather(erows, [jf, jnp.full((16,), 128, jnp.int32) + ci])
            e2 = plsc.load_gather(erows, [jf, jnp.full((16,), 256, jnp.int32) + ci])
            e3 = plsc.load_gather(erows, [jf, jnp.full((16,), 384, jnp.int32) + ci])
            for t in range(NT):
                cs = pl.ds(t * 16, 16)
                v = brows[j, cs] + e0 * w3r[0][t]
                v = v + e1 * w3r[1][t] + e2 * w3r[2][t] + e3 * w3r[3][t]
                acc[r, cs] = jnp.maximum(acc[r, cs], v)
            return 0

        lax.fori_loop(0, n, edge_body, 0)
        return 0

    lax.fori_loop(0, nchunks, chunk_body, 0)

    def fin_body(i, _):
        r = i // NT
        t = i % NT
        cs = pl.ds(t * 16, 16)
        v = acc[r, cs]
        res = jnp.maximum(arows[r, cs] + v, 0.0)
        acc[r, cs] = jnp.where(v == neg, v, res)
        return 0

    lax.fori_loop(0, RPW * NT, fin_body, 0)

    pltpu.sync_copy(acc, out_hbm.at[pl.ds(r0, RPW)])


_sc_edge_kernel = pl.kernel(
    _sc_body,
    out_type=jax.ShapeDtypeStruct((V, C), jnp.float32),
    mesh=_sc_mesh,
    compiler_params=pltpu.CompilerParams(
        needs_layout_passes=False, use_tc_tiling_on_sc=False),
    scratch_types=_SC_SCRATCH,
)


def kernel(adj_src, adj_dst, adj_values, node_features, edge_attributes, W, b):
    del adj_values
    a_proj, b_proj = _proj(node_features, W, b.reshape(1, C))
    rs = jnp.full((V + 1,), E, dtype=jnp.int32).at[adj_src].min(
        jnp.arange(E, dtype=jnp.int32))
    rs = lax.cummin(rs, axis=0, reverse=True)
    rs = jnp.concatenate([rs, jnp.full((15,), E, dtype=jnp.int32)])
    pad = jnp.zeros((CHUNK + 8,), dtype=jnp.int32)
    src_p = jnp.concatenate([adj_src, pad])
    dst_p = jnp.concatenate([adj_dst, pad])
    e4 = edge_attributes.reshape(1, V, 16, 128, CE).transpose(
        0, 1, 2, 4, 3).reshape(V * 16, CE * 128)
    w3 = W[2 * C:, :]
    return _sc_edge_kernel(rs, src_p, dst_p, a_proj, b_proj, e4, w3)


# speedup vs baseline: 21.8611x; 1.4057x over previous
import functools

import jax
import jax.numpy as jnp
from jax import lax
from jax.experimental import pallas as pl
from jax.experimental.pallas import tpu as pltpu
from jax.experimental.pallas import tpu_sc as plsc

V = 2048
E = 65536
C = 128
CE = 4
H = 128

NW = 32
RPW = V // NW
CHUNK = 128
NT = C // 16


def _proj_body(nf_ref, w_ref, b_ref, a_ref, b2_ref):
    nf = nf_ref[...]
    w1 = w_ref[0:C, :]
    w2 = w_ref[C:2 * C, :]
    a_ref[...] = (
        jnp.dot(nf, w1 - w2, preferred_element_type=jnp.float32) + b_ref[...]
    )
    b2_ref[...] = jnp.dot(nf, w2, preferred_element_type=jnp.float32)


_proj = pl.pallas_call(
    _proj_body,
    out_shape=(
        jax.ShapeDtypeStruct((V, C), jnp.float32),
        jax.ShapeDtypeStruct((V, C), jnp.float32),
    ),
)


_sc_mesh = plsc.VectorSubcoreMesh(
    core_axis_name="c", subcore_axis_name="s", num_cores=2, num_subcores=16)


_SC_SCRATCH = [
    pltpu.VMEM((RPW + 16,), jnp.int32),
    pltpu.VMEM((CHUNK + 24,), jnp.int32),
    pltpu.VMEM((CHUNK + 24,), jnp.int32),
    pltpu.VMEM((CHUNK,), jnp.int32),
    pltpu.VMEM((CHUNK,), jnp.int32),
    pltpu.VMEM((CHUNK, C), jnp.float32),
    pltpu.VMEM((CHUNK, 512), jnp.float32),
    pltpu.VMEM((RPW, C), jnp.float32),
    pltpu.VMEM((CE, C), jnp.float32),
    pltpu.VMEM((RPW, C), jnp.float32),
    pltpu.SemaphoreType.DMA,
    pltpu.SemaphoreType.DMA,
]


def _sc_body(rs_hbm, src_hbm, dst_hbm, a_hbm, b2_hbm, e4_hbm, w3_hbm,
                    out_hbm, rs_v, srcb, dstb, idxb, idxe, brows, erows,
                    arows, w3v, acc, sem, sem2):
    wid = lax.axis_index("s") * 2 + lax.axis_index("c")
    r0 = wid * RPW

    pltpu.sync_copy(rs_hbm.at[pl.ds(r0, RPW + 16)], rs_v)
    pltpu.sync_copy(a_hbm.at[pl.ds(r0, RPW)], arows)
    pltpu.sync_copy(w3_hbm, w3v)
    w3r = [[w3v[k, pl.ds(t * 16, 16)] for t in range(NT)] for k in range(CE)]

    neg = jnp.full((16,), -jnp.inf, dtype=jnp.float32)

    def init_body(i, _):
        r = i // NT
        t = i % NT
        acc[r, pl.ds(t * 16, 16)] = neg
        return 0

    lax.fori_loop(0, RPW * NT, init_body, 0)

    s = rs_v[pl.ds(0, 16)][0]
    cnt = rs_v[pl.ds(RPW, 16)][0] - s
    off = s % 8
    sa = s - off
    nchunks = (cnt + CHUNK - 1) // CHUNK

    def chunk_body(k, _):
        base = pl.multiple_of(sa + k * CHUNK, 8)
        pltpu.sync_copy(src_hbm.at[pl.ds(base, CHUNK + 8)],
                        srcb.at[pl.ds(0, CHUNK + 8)])
        pltpu.sync_copy(dst_hbm.at[pl.ds(base, CHUNK + 8)],
                        dstb.at[pl.ds(0, CHUNK + 8)])
        lanes = jnp.arange(16, dtype=jnp.int32)
        for t in range(CHUNK // 16):
            ii = lanes + (off + t * 16)
            sv = plsc.load_gather(srcb, [ii])
            dv = plsc.load_gather(dstb, [ii])
            idxb[pl.ds(t * 16, 16)] = dv
            idxe[pl.ds(t * 16, 16)] = sv * 16 + (dv >> 7)
        cpb = pltpu.async_copy(b2_hbm.at[idxb], brows, sem)
        cpe = pltpu.async_copy(e4_hbm.at[idxe], erows, sem2)
        cpb.wait()
        cpe.wait()
        n = jnp.minimum(cnt - k * CHUNK, CHUNK)

        def run_body(j):
            r = srcb[pl.ds(off + j, 16)][0]

            def inner_cond(ist):
                jj = ist[0]
                return (jj < n) & (srcb[pl.ds(off + jj, 16)][0] == r)

            def inner_body(ist):
                jj = ist[0]
                accs = ist[1:]
                ci = dstb[pl.ds(off + jj, 16)][0] & 127
                jf = jnp.full((16,), jj, dtype=jnp.int32)
                e0 = plsc.load_gather(
                    erows, [jf, jnp.full((16,), 0, jnp.int32) + ci])
                e1 = plsc.load_gather(
                    erows, [jf, jnp.full((16,), 128, jnp.int32) + ci])
                e2 = plsc.load_gather(
                    erows, [jf, jnp.full((16,), 256, jnp.int32) + ci])
                e3 = plsc.load_gather(
                    erows, [jf, jnp.full((16,), 384, jnp.int32) + ci])
                new = []
                for t in range(NT):
                    cs = pl.ds(t * 16, 16)
                    v = brows[jj, cs] + e0 * w3r[0][t]
                    v = v + e1 * w3r[1][t] + e2 * w3r[2][t] + e3 * w3r[3][t]
                    new.append(jnp.maximum(accs[t], v))
                return (jj + 1,) + tuple(new)

            rr = r - r0
            accs0 = tuple(acc[rr, pl.ds(t * 16, 16)] for t in range(NT))
            out = lax.while_loop(inner_cond, inner_body, (j,) + accs0)
            for t in range(NT):
                acc[rr, pl.ds(t * 16, 16)] = out[1 + t]
            return out[0]

        lax.while_loop(lambda j: j < n, run_body, jnp.int32(0))
        return 0

    lax.fori_loop(0, nchunks, chunk_body, 0)

    def fin_body(i, _):
        r = i // NT
        t = i % NT
        cs = pl.ds(t * 16, 16)
        v = acc[r, cs]
        res = jnp.maximum(arows[r, cs] + v, 0.0)
        acc[r, cs] = jnp.where(v == neg, v, res)
        return 0

    lax.fori_loop(0, RPW * NT, fin_body, 0)

    pltpu.sync_copy(acc, out_hbm.at[pl.ds(r0, RPW)])


_sc_edge_kernel = pl.kernel(
    _sc_body,
    out_type=jax.ShapeDtypeStruct((V, C), jnp.float32),
    mesh=_sc_mesh,
    compiler_params=pltpu.CompilerParams(
        needs_layout_passes=False, use_tc_tiling_on_sc=False),
    scratch_types=_SC_SCRATCH,
)


def kernel(adj_src, adj_dst, adj_values, node_features, edge_attributes, W, b):
    del adj_values
    a_proj, b_proj = _proj(node_features, W, b.reshape(1, C))
    rs = jnp.full((V + 1,), E, dtype=jnp.int32).at[adj_src].min(
        jnp.arange(E, dtype=jnp.int32))
    rs = lax.cummin(rs, axis=0, reverse=True)
    rs = jnp.concatenate([rs, jnp.full((15,), E, dtype=jnp.int32)])
    pad = jnp.zeros((CHUNK + 8,), dtype=jnp.int32)
    src_p = jnp.concatenate([adj_src, pad])
    dst_p = jnp.concatenate([adj_dst, pad])
    e4 = edge_attributes.reshape(1, V, 16, 128, CE).transpose(
        0, 1, 2, 4, 3).reshape(V * 16, CE * 128)
    w3 = W[2 * C:, :]
    return _sc_edge_kernel(rs, src_p, dst_p, a_proj, b_proj, e4, w3)


# speedup vs baseline: 26.5378x; 1.2139x over previous
import functools

import jax
import jax.numpy as jnp
from jax import lax
from jax.experimental import pallas as pl
from jax.experimental.pallas import tpu as pltpu
from jax.experimental.pallas import tpu_sc as plsc

V = 2048
E = 65536
C = 128
CE = 4
H = 128

NW = 32
RPW = V // NW
CHUNK = 128
NT = C // 16


def _proj_body(nf_ref, w_ref, b_ref, a_ref, b2_ref):
    nf = nf_ref[...]
    w1 = w_ref[0:C, :]
    w2 = w_ref[C:2 * C, :]
    a_ref[...] = (
        jnp.dot(nf, w1 - w2, preferred_element_type=jnp.float32) + b_ref[...]
    )
    b2_ref[...] = jnp.dot(nf, w2, preferred_element_type=jnp.float32)


_proj = pl.pallas_call(
    _proj_body,
    out_shape=(
        jax.ShapeDtypeStruct((V, C), jnp.float32),
        jax.ShapeDtypeStruct((V, C), jnp.float32),
    ),
)


_sc_mesh = plsc.VectorSubcoreMesh(
    core_axis_name="c", subcore_axis_name="s", num_cores=2, num_subcores=16)


_SC_SCRATCH = [
    pltpu.VMEM((48,), jnp.int32),
    pltpu.VMEM((CHUNK + 24,), jnp.int32),
    pltpu.VMEM((CHUNK + 24,), jnp.int32),
    pltpu.VMEM((CHUNK,), jnp.int32),
    pltpu.VMEM((CHUNK,), jnp.int32),
    pltpu.VMEM((CHUNK, C), jnp.float32),
    pltpu.VMEM((CHUNK, 512), jnp.float32),
    pltpu.VMEM((RPW, C), jnp.float32),
    pltpu.VMEM((CE, C), jnp.float32),
    pltpu.VMEM((RPW, C), jnp.float32),
    pltpu.SemaphoreType.DMA,
    pltpu.SemaphoreType.DMA,
]


def _sc_body(rs_hbm, src_hbm, dst_hbm, a_hbm, b2_hbm, e4_hbm, w3_hbm,
                    out_hbm, rs_v, srcb, dstb, idxb, idxe, brows, erows,
                    arows, w3v, acc, sem, sem2):
    wid = lax.axis_index("s") * 2 + lax.axis_index("c")
    r0 = wid * RPW

    pltpu.sync_copy(rs_hbm, rs_v)
    pltpu.sync_copy(a_hbm.at[pl.ds(r0, RPW)], arows)
    pltpu.sync_copy(w3_hbm, w3v)
    w3r = [[w3v[k, pl.ds(t * 16, 16)] for t in range(NT)] for k in range(CE)]

    neg = jnp.full((16,), -jnp.inf, dtype=jnp.float32)

    def init_body(i, _):
        r = i // NT
        t = i % NT
        acc[r, pl.ds(t * 16, 16)] = neg
        return 0

    lax.fori_loop(0, RPW * NT, init_body, 0)

    s = rs_v[pl.ds(wid, 16)][0]
    cnt = rs_v[pl.ds(wid + 1, 16)][0] - s
    off = s % 8
    sa = s - off
    nchunks = (cnt + CHUNK - 1) // CHUNK

    def chunk_body(k, _):
        base = pl.multiple_of(sa + k * CHUNK, 8)
        pltpu.sync_copy(src_hbm.at[pl.ds(base, CHUNK + 8)],
                        srcb.at[pl.ds(0, CHUNK + 8)])
        pltpu.sync_copy(dst_hbm.at[pl.ds(base, CHUNK + 8)],
                        dstb.at[pl.ds(0, CHUNK + 8)])
        lanes = jnp.arange(16, dtype=jnp.int32)
        for t in range(CHUNK // 16):
            ii = lanes + (off + t * 16)
            sv = plsc.load_gather(srcb, [ii])
            dv = plsc.load_gather(dstb, [ii])
            idxb[pl.ds(t * 16, 16)] = dv
            idxe[pl.ds(t * 16, 16)] = sv * 16 + (dv >> 7)
        cpb = pltpu.async_copy(b2_hbm.at[idxb], brows, sem)
        cpe = pltpu.async_copy(e4_hbm.at[idxe], erows, sem2)
        cpb.wait()
        cpe.wait()
        n = jnp.minimum(cnt - k * CHUNK, CHUNK)

        def run_body(j):
            r = srcb[pl.ds(off + j, 16)][0]

            def inner_cond(ist):
                jj = ist[0]
                return (jj < n) & (srcb[pl.ds(off + jj, 16)][0] == r)

            def inner_body(ist):
                jj = ist[0]
                accs = ist[1:]
                ci = dstb[pl.ds(off + jj, 16)][0] & 127
                jf = jnp.full((16,), jj, dtype=jnp.int32)
                e0 = plsc.load_gather(
                    erows, [jf, jnp.full((16,), 0, jnp.int32) + ci])
                e1 = plsc.load_gather(
                    erows, [jf, jnp.full((16,), 128, jnp.int32) + ci])
                e2 = plsc.load_gather(
                    erows, [jf, jnp.full((16,), 256, jnp.int32) + ci])
                e3 = plsc.load_gather(
                    erows, [jf, jnp.full((16,), 384, jnp.int32) + ci])
                new = []
                for t in range(NT):
                    cs = pl.ds(t * 16, 16)
                    v = brows[jj, cs] + e0 * w3r[0][t]
                    v = v + e1 * w3r[1][t] + e2 * w3r[2][t] + e3 * w3r[3][t]
                    new.append(jnp.maximum(accs[t], v))
                return (jj + 1,) + tuple(new)

            rr = r - r0
            accs0 = tuple(acc[rr, pl.ds(t * 16, 16)] for t in range(NT))
            out = lax.while_loop(inner_cond, inner_body, (j,) + accs0)
            for t in range(NT):
                acc[rr, pl.ds(t * 16, 16)] = out[1 + t]
            return out[0]

        lax.while_loop(lambda j: j < n, run_body, jnp.int32(0))
        return 0

    lax.fori_loop(0, nchunks, chunk_body, 0)

    def fin_body(i, _):
        r = i // NT
        t = i % NT
        cs = pl.ds(t * 16, 16)
        v = acc[r, cs]
        res = jnp.maximum(arows[r, cs] + v, 0.0)
        acc[r, cs] = jnp.where(v == neg, v, res)
        return 0

    lax.fori_loop(0, RPW * NT, fin_body, 0)

    pltpu.sync_copy(acc, out_hbm.at[pl.ds(r0, RPW)])


_sc_edge_kernel = pl.kernel(
    _sc_body,
    out_type=jax.ShapeDtypeStruct((V, C), jnp.float32),
    mesh=_sc_mesh,
    compiler_params=pltpu.CompilerParams(
        needs_layout_passes=False, use_tc_tiling_on_sc=False),
    scratch_types=_SC_SCRATCH,
)


def kernel(adj_src, adj_dst, adj_values, node_features, edge_attributes, W, b):
    del adj_values
    a_proj, b_proj = _proj(node_features, W, b.reshape(1, C))
    rs = jnp.searchsorted(
        adj_src, jnp.arange(0, V + 1, RPW, dtype=jnp.int32)
    ).astype(jnp.int32)
    rs = jnp.concatenate([rs, jnp.full((15,), E, dtype=jnp.int32)])
    pad = jnp.zeros((CHUNK + 8,), dtype=jnp.int32)
    src_p = jnp.concatenate([adj_src, pad])
    dst_p = jnp.concatenate([adj_dst, pad])
    e4 = edge_attributes.reshape(1, V, 16, 128, CE).transpose(
        0, 1, 2, 4, 3).reshape(V * 16, CE * 128)
    w3 = W[2 * C:, :]
    return _sc_edge_kernel(rs, src_p, dst_p, a_proj, b_proj, e4, w3)


# speedup vs baseline: 27.4691x; 1.0351x over previous
import functools

import jax
import jax.numpy as jnp
from jax import lax
from jax.experimental import pallas as pl
from jax.experimental.pallas import tpu as pltpu
from jax.experimental.pallas import tpu_sc as plsc

V = 2048
E = 65536
C = 128
CE = 4
H = 128

NW = 32
RPW = V // NW
CHUNK = 128
NT = C // 16


def _proj_body(nf_ref, w_ref, b_ref, a_ref, b2_ref):
    nf = nf_ref[...]
    w1 = w_ref[0:C, :]
    w2 = w_ref[C:2 * C, :]
    a_ref[...] = (
        jnp.dot(nf, w1 - w2, preferred_element_type=jnp.float32) + b_ref[...]
    )
    b2_ref[...] = jnp.dot(nf, w2, preferred_element_type=jnp.float32)


_proj = pl.pallas_call(
    _proj_body,
    out_shape=(
        jax.ShapeDtypeStruct((V, C), jnp.float32),
        jax.ShapeDtypeStruct((V, C), jnp.float32),
    ),
)


_sc_mesh = plsc.VectorSubcoreMesh(
    core_axis_name="c", subcore_axis_name="s", num_cores=2, num_subcores=16)


_SC_SCRATCH = [
    pltpu.VMEM((48,), jnp.int32),
    pltpu.VMEM((CHUNK + 24,), jnp.int32),
    pltpu.VMEM((CHUNK + 24,), jnp.int32),
    pltpu.VMEM((CHUNK,), jnp.int32),
    pltpu.VMEM((CHUNK,), jnp.int32),
    pltpu.VMEM((CHUNK, C), jnp.float32),
    pltpu.VMEM((CHUNK, 512), jnp.float32),
    pltpu.VMEM((RPW, C), jnp.float32),
    pltpu.VMEM((CE, C), jnp.float32),
    pltpu.VMEM((RPW, C), jnp.float32),
    pltpu.SemaphoreType.DMA,
    pltpu.SemaphoreType.DMA,
]


def _sc_body(rs_hbm, src_hbm, dst_hbm, a_hbm, b2_hbm, e4_hbm, w3_hbm,
                    out_hbm, rs_v, srcb, dstb, idxb, idxe, brows, erows,
                    arows, w3v, acc, sem, sem2):
    wid = lax.axis_index("s") * 2 + lax.axis_index("c")
    r0 = wid * RPW

    pltpu.sync_copy(rs_hbm, rs_v)
    pltpu.sync_copy(a_hbm.at[pl.ds(r0, RPW)], arows)
    pltpu.sync_copy(w3_hbm, w3v)
    w3r = [[w3v[k, pl.ds(t * 16, 16)] for t in range(NT)] for k in range(CE)]

    neg = jnp.full((16,), -jnp.inf, dtype=jnp.float32)

    def init_body(i, _):
        r = i // NT
        t = i % NT
        acc[r, pl.ds(t * 16, 16)] = neg
        return 0

    lax.fori_loop(0, RPW * NT, init_body, 0)

    s = rs_v[pl.ds(wid, 16)][0]
    cnt = rs_v[pl.ds(wid + 1, 16)][0] - s
    off = s % 8
    sa = s - off
    nchunks = (cnt + CHUNK - 1) // CHUNK

    def chunk_body(k, _):
        base = pl.multiple_of(sa + k * CHUNK, 8)
        cps = pltpu.async_copy(src_hbm.at[pl.ds(base, CHUNK + 8)],
                               srcb.at[pl.ds(0, CHUNK + 8)], sem)
        cpd = pltpu.async_copy(dst_hbm.at[pl.ds(base, CHUNK + 8)],
                               dstb.at[pl.ds(0, CHUNK + 8)], sem2)
        cps.wait()
        cpd.wait()
        lanes = jnp.arange(16, dtype=jnp.int32)
        for t in range(CHUNK // 16):
            ii = lanes + (off + t * 16)
            sv = plsc.load_gather(srcb, [ii])
            dv = plsc.load_gather(dstb, [ii])
            idxb[pl.ds(t * 16, 16)] = dv
            idxe[pl.ds(t * 16, 16)] = sv * 16 + (dv >> 7)
        cpb = pltpu.async_copy(b2_hbm.at[idxb], brows, sem)
        cpe = pltpu.async_copy(e4_hbm.at[idxe], erows, sem2)
        cpb.wait()
        cpe.wait()
        n = jnp.minimum(cnt - k * CHUNK, CHUNK)

        def run_body(j):
            r = srcb[pl.ds(off + j, 16)][0]

            def inner_cond(ist):
                jj = ist[0]
                return (jj < n) & (srcb[pl.ds(off + jj, 16)][0] == r)

            def inner_body(ist):
                jj = ist[0]
                accs = ist[1:]
                ci = dstb[pl.ds(off + jj, 16)][0] & 127
                jf = jnp.full((16,), jj, dtype=jnp.int32)
                e0 = plsc.load_gather(
                    erows, [jf, jnp.full((16,), 0, jnp.int32) + ci])
                e1 = plsc.load_gather(
                    erows, [jf, jnp.full((16,), 128, jnp.int32) + ci])
                e2 = plsc.load_gather(
                    erows, [jf, jnp.full((16,), 256, jnp.int32) + ci])
                e3 = plsc.load_gather(
                    erows, [jf, jnp.full((16,), 384, jnp.int32) + ci])
                new = []
                for t in range(NT):
                    cs = pl.ds(t * 16, 16)
                    v = brows[jj, cs] + e0 * w3r[0][t]
                    v = v + e1 * w3r[1][t] + e2 * w3r[2][t] + e3 * w3r[3][t]
                    new.append(jnp.maximum(accs[t], v))
                return (jj + 1,) + tuple(new)

            rr = r - r0
            accs0 = tuple(acc[rr, pl.ds(t * 16, 16)] for t in range(NT))
            out = lax.while_loop(inner_cond, inner_body, (j,) + accs0)
            for t in range(NT):
                acc[rr, pl.ds(t * 16, 16)] = out[1 + t]
            return out[0]

        lax.while_loop(lambda j: j < n, run_body, jnp.int32(0))
        return 0

    lax.fori_loop(0, nchunks, chunk_body, 0)

    def fin_body(i, _):
        r = i // NT
        t = i % NT
        cs = pl.ds(t * 16, 16)
        v = acc[r, cs]
        res = jnp.maximum(arows[r, cs] + v, 0.0)
        acc[r, cs] = jnp.where(v == neg, v, res)
        return 0

    lax.fori_loop(0, RPW * NT, fin_body, 0)

    pltpu.sync_copy(acc, out_hbm.at[pl.ds(r0, RPW)])


_sc_edge_kernel = pl.kernel(
    _sc_body,
    out_type=jax.ShapeDtypeStruct((V, C), jnp.float32),
    mesh=_sc_mesh,
    compiler_params=pltpu.CompilerParams(
        needs_layout_passes=False, use_tc_tiling_on_sc=False),
    scratch_types=_SC_SCRATCH,
)


def kernel(adj_src, adj_dst, adj_values, node_features, edge_attributes, W, b):
    del adj_values
    a_proj, b_proj = _proj(node_features, W, b.reshape(1, C))
    rs = jnp.searchsorted(
        adj_src, jnp.arange(0, V + 1, RPW, dtype=jnp.int32)
    ).astype(jnp.int32)
    rs = jnp.concatenate([rs, jnp.full((15,), E, dtype=jnp.int32)])
    pad = jnp.zeros((CHUNK + 8,), dtype=jnp.int32)
    src_p = jnp.concatenate([adj_src, pad])
    dst_p = jnp.concatenate([adj_dst, pad])
    e4 = edge_attributes.reshape(1, V, 16, 128, CE).transpose(
        0, 1, 2, 4, 3).reshape(V * 16, CE * 128)
    w3 = W[2 * C:, :]
    return _sc_edge_kernel(rs, src_p, dst_p, a_proj, b_proj, e4, w3)


# speedup vs baseline: 27.5023x; 1.0012x over previous
import jax
import jax.numpy as jnp
from jax import lax
from jax.experimental import pallas as pl
from jax.experimental.pallas import tpu as pltpu
from jax.experimental.pallas import tpu_sc as plsc

V = 2048
E = 65536
C = 128
CE = 4
H = 128

NW = 32
RPW = V // NW
CHUNK = 128
NT = C // 16


def _proj_body(nf_ref, w_ref, b_ref, a_ref, b2_ref):
    nf = nf_ref[...]
    w1 = w_ref[0:C, :]
    w2 = w_ref[C:2 * C, :]
    a_ref[...] = (
        jnp.dot(nf, w1 - w2, preferred_element_type=jnp.float32) + b_ref[...]
    )
    b2_ref[...] = jnp.dot(nf, w2, preferred_element_type=jnp.float32)


_proj = pl.pallas_call(
    _proj_body,
    out_shape=(
        jax.ShapeDtypeStruct((V, C), jnp.float32),
        jax.ShapeDtypeStruct((V, C), jnp.float32),
    ),
)


_sc_mesh = plsc.VectorSubcoreMesh(
    core_axis_name="c", subcore_axis_name="s", num_cores=2, num_subcores=16)


_SC_SCRATCH = [
    pltpu.VMEM((48,), jnp.int32),
    pltpu.VMEM((CHUNK + 24,), jnp.int32),
    pltpu.VMEM((CHUNK + 24,), jnp.int32),
    pltpu.VMEM((CHUNK,), jnp.int32),
    pltpu.VMEM((CHUNK,), jnp.int32),
    pltpu.VMEM((CHUNK, C), jnp.float32),
    pltpu.VMEM((CHUNK, 512), jnp.float32),
    pltpu.VMEM((RPW, C), jnp.float32),
    pltpu.VMEM((CE, C), jnp.float32),
    pltpu.VMEM((RPW, C), jnp.float32),
    pltpu.SemaphoreType.DMA,
    pltpu.SemaphoreType.DMA,
]


def _sc_body(rs_hbm, src_hbm, dst_hbm, a_hbm, b2_hbm, e4_hbm, w3_hbm,
                    out_hbm, rs_v, srcb, dstb, idxb, idxe, brows, erows,
                    arows, w3v, acc, sem, sem2):
    wid = lax.axis_index("s") * 2 + lax.axis_index("c")
    r0 = wid * RPW

    pltpu.sync_copy(rs_hbm, rs_v)
    pltpu.sync_copy(a_hbm.at[pl.ds(r0, RPW)], arows)
    pltpu.sync_copy(w3_hbm, w3v)
    w3r = [[w3v[k, pl.ds(t * 16, 16)] for t in range(NT)] for k in range(CE)]

    neg = jnp.full((16,), -jnp.inf, dtype=jnp.float32)

    def init_body(i, _):
        r = i // NT
        t = i % NT
        acc[r, pl.ds(t * 16, 16)] = neg
        return 0

    lax.fori_loop(0, RPW * NT, init_body, 0)

    s = rs_v[pl.ds(wid, 16)][0]
    cnt = rs_v[pl.ds(wid + 1, 16)][0] - s
    off = s % 8
    sa = s - off
    nchunks = (cnt + CHUNK - 1) // CHUNK

    def chunk_body(k, _):
        base = pl.multiple_of(sa + k * CHUNK, 8)
        cps = pltpu.async_copy(src_hbm.at[pl.ds(base, CHUNK + 8)],
                               srcb.at[pl.ds(0, CHUNK + 8)], sem)
        cpd = pltpu.async_copy(dst_hbm.at[pl.ds(base, CHUNK + 8)],
                               dstb.at[pl.ds(0, CHUNK + 8)], sem2)
        cps.wait()
        cpd.wait()
        lanes = jnp.arange(16, dtype=jnp.int32)
        for t in range(CHUNK // 16):
            ii = lanes + (off + t * 16)
            sv = plsc.load_gather(srcb, [ii])
            dv = plsc.load_gather(dstb, [ii])
            idxb[pl.ds(t * 16, 16)] = dv
            idxe[pl.ds(t * 16, 16)] = sv * 16 + (dv >> 7)
        cpb = pltpu.async_copy(b2_hbm.at[idxb], brows, sem)
        cpe = pltpu.async_copy(e4_hbm.at[idxe], erows, sem2)
        cpb.wait()
        cpe.wait()
        n = jnp.minimum(cnt - k * CHUNK, CHUNK)

        def run_body(j):
            r = srcb[pl.ds(off + j, 16)][0]

            def inner_cond(ist):
                jj = ist[0]
                return (jj < n) & (srcb[pl.ds(off + jj, 16)][0] == r)

            def inner_body(ist):
                jj = ist[0]
                accs = ist[1:]
                ci = dstb[pl.ds(off + jj, 16)][0] & 127
                jf = jnp.full((16,), jj, dtype=jnp.int32)
                e0 = plsc.load_gather(
                    erows, [jf, jnp.full((16,), 0, jnp.int32) + ci])
                e1 = plsc.load_gather(
                    erows, [jf, jnp.full((16,), 128, jnp.int32) + ci])
                e2 = plsc.load_gather(
                    erows, [jf, jnp.full((16,), 256, jnp.int32) + ci])
                e3 = plsc.load_gather(
                    erows, [jf, jnp.full((16,), 384, jnp.int32) + ci])
                new = []
                for t in range(NT):
                    cs = pl.ds(t * 16, 16)
                    v = brows[jj, cs] + e0 * w3r[0][t]
                    v = v + e1 * w3r[1][t] + e2 * w3r[2][t] + e3 * w3r[3][t]
                    new.append(jnp.maximum(accs[t], v))
                return (jj + 1,) + tuple(new)

            rr = r - r0
            accs0 = tuple(acc[rr, pl.ds(t * 16, 16)] for t in range(NT))
            out = lax.while_loop(inner_cond, inner_body, (j,) + accs0)
            for t in range(NT):
                acc[rr, pl.ds(t * 16, 16)] = out[1 + t]
            return out[0]

        lax.while_loop(lambda j: j < n, run_body, jnp.int32(0))
        return 0

    lax.fori_loop(0, nchunks, chunk_body, 0)

    def fin_body(i, _):
        r = i // NT
        t = i % NT
        cs = pl.ds(t * 16, 16)
        v = acc[r, cs]
        res = jnp.maximum(arows[r, cs] + v, 0.0)
        acc[r, cs] = jnp.where(v == neg, v, res)
        return 0

    lax.fori_loop(0, RPW * NT, fin_body, 0)

    pltpu.sync_copy(acc, out_hbm.at[pl.ds(r0, RPW)])


_sc_edge_kernel = pl.kernel(
    _sc_body,
    out_type=jax.ShapeDtypeStruct((V, C), jnp.float32),
    mesh=_sc_mesh,
    compiler_params=pltpu.CompilerParams(
        needs_layout_passes=False, use_tc_tiling_on_sc=False),
    scratch_types=_SC_SCRATCH,
)


def kernel(adj_src, adj_dst, adj_values, node_features, edge_attributes, W, b):
    del adj_values
    a_proj, b_proj = _proj(node_features, W, b.reshape(1, C))
    rs = jnp.searchsorted(
        adj_src, jnp.arange(0, V + 1, RPW, dtype=jnp.int32)
    ).astype(jnp.int32)
    rs = jnp.concatenate([rs, jnp.full((15,), E, dtype=jnp.int32)])
    pad = jnp.zeros((CHUNK + 8,), dtype=jnp.int32)
    src_p = jnp.concatenate([adj_src, pad])
    dst_p = jnp.concatenate([adj_dst, pad])
    e4 = edge_attributes.reshape(1, V, 16, 128, CE).transpose(
        0, 1, 2, 4, 3).reshape(V * 16, CE * 128)
    w3 = W[2 * C:, :]
    return _sc_edge_kernel(rs, src_p, dst_p, a_proj, b_proj, e4, w3)
